# Initial kernel scaffold; baseline (speedup 1.0000x reference)
import jax
import jax.numpy as jnp
from jax.experimental import pallas as pl


def kernel(inputs, W, b, noise):
    raise NotImplementedError("write your pallas kernel here")


# speedup vs baseline: 7.1431x; 7.1431x over previous
import math

import jax
import jax.numpy as jnp
from jax.experimental import pallas as pl

B = 16384
DIM = 128
BLOCK = 2048

_HALF_LOG_2PI = 0.5 * math.log(2.0 * math.pi)


def _fused_kernel(x_ref, nz_ref, wm_ref, wv_ref, bm_ref, bv_ref, v_ref, o2_ref):
    x = x_ref[...]
    nz = nz_ref[...]

    mean = jnp.dot(x, wm_ref[...], preferred_element_type=jnp.float32) + bm_ref[...]
    log_var = jnp.dot(x, wv_ref[...], preferred_element_type=jnp.float32) + bv_ref[...]

    std = jnp.exp(0.5 * log_var)
    u = nz * std + mean

    mx = jnp.max(x, axis=1, keepdims=True)
    iota = jax.lax.broadcasted_iota(jnp.int32, x.shape, 1)
    idx = jnp.min(jnp.where(x == mx, iota, DIM), axis=1, keepdims=True)
    mask = iota == idx

    T = jnp.sum(jnp.where(mask, u, 0.0), axis=1, keepdims=True)

    d = T - u
    sp = jnp.maximum(d, 0.0) + jnp.log1p(jnp.exp(-jnp.abs(d)))

    v_ref[...] = jnp.where(mask, T, T - sp)

    log_det = jnp.sum(jnp.where(mask, 0.0, d - sp), axis=1, keepdims=True)
    log_pu = -0.5 * jnp.sum(nz * nz, axis=1, keepdims=True) - DIM * _HALF_LOG_2PI
    o2_ref[...] = log_det - log_pu


def kernel(inputs, W, b, noise):
    wm_t = W[:DIM].T
    wv_t = W[DIM:].T
    bm = b[:DIM].reshape(1, DIM)
    bv = b[DIM:].reshape(1, DIM)

    n = inputs.shape[0]
    grid = (n // BLOCK,)

    row_spec = pl.BlockSpec((BLOCK, DIM), lambda i: (i, 0))
    full_spec = pl.BlockSpec((DIM, DIM), lambda i: (0, 0))
    bias_spec = pl.BlockSpec((1, DIM), lambda i: (0, 0))

    v, o2 = pl.pallas_call(
        _fused_kernel,
        grid=grid,
        in_specs=[row_spec, row_spec, full_spec, full_spec, bias_spec, bias_spec],
        out_specs=[
            pl.BlockSpec((BLOCK, DIM), lambda i: (i, 0)),
            pl.BlockSpec((BLOCK, 1), lambda i: (i, 0)),
        ],
        out_shape=[
            jax.ShapeDtypeStruct((n, DIM), jnp.float32),
            jax.ShapeDtypeStruct((n, 1), jnp.float32),
        ],
    )(inputs, noise, wm_t, wv_t, bm, bv)
    return (v, o2)


# speedup vs baseline: 7.4286x; 1.0400x over previous
import math

import jax
import jax.numpy as jnp
from jax.experimental import pallas as pl

B = 16384
DIM = 128
BLOCK = 2048

_HALF_LOG_2PI = 0.5 * math.log(2.0 * math.pi)
_LOG2 = math.log(2.0)


def _fused_kernel(x_ref, nz_ref, wm_ref, wv_ref, bm_ref, bv_ref, v_ref, o2_ref):
    x = x_ref[...]
    nz = nz_ref[...]

    xb = x.astype(jnp.bfloat16)
    mean = jnp.dot(xb, wm_ref[...], preferred_element_type=jnp.float32) + bm_ref[...]
    log_var = jnp.dot(xb, wv_ref[...], preferred_element_type=jnp.float32) + bv_ref[...]

    std = jnp.exp(0.5 * log_var)
    u = nz * std + mean

    mx = jnp.max(x, axis=1, keepdims=True)
    iota = jax.lax.broadcasted_iota(jnp.int32, x.shape, 1)
    idx = jnp.min(jnp.where(x == mx, iota, DIM), axis=1, keepdims=True)
    mask = iota == idx

    T = jnp.sum(jnp.where(mask, u, 0.0), axis=1, keepdims=True)

    d = T - u
    sp = jnp.maximum(d, 0.0) + jnp.log1p(jnp.exp(-jnp.abs(d)))

    v_ref[...] = jnp.where(mask, T, T - sp)

    log_det = jnp.sum(d - sp, axis=1, keepdims=True) + _LOG2
    log_pu = -0.5 * jnp.sum(nz * nz, axis=1, keepdims=True) - DIM * _HALF_LOG_2PI
    o2_ref[...] = log_det - log_pu


def kernel(inputs, W, b, noise):
    wm_t = W[:DIM].T.astype(jnp.bfloat16)
    wv_t = W[DIM:].T.astype(jnp.bfloat16)
    bm = b[:DIM].reshape(1, DIM)
    bv = b[DIM:].reshape(1, DIM)

    n = inputs.shape[0]
    grid = (n // BLOCK,)

    row_spec = pl.BlockSpec((BLOCK, DIM), lambda i: (i, 0))
    full_spec = pl.BlockSpec((DIM, DIM), lambda i: (0, 0))
    bias_spec = pl.BlockSpec((1, DIM), lambda i: (0, 0))

    v, o2 = pl.pallas_call(
        _fused_kernel,
        grid=grid,
        in_specs=[row_spec, row_spec, full_spec, full_spec, bias_spec, bias_spec],
        out_specs=[
            pl.BlockSpec((BLOCK, DIM), lambda i: (i, 0)),
            pl.BlockSpec((BLOCK, 1), lambda i: (i, 0)),
        ],
        out_shape=[
            jax.ShapeDtypeStruct((n, DIM), jnp.float32),
            jax.ShapeDtypeStruct((n, 1), jnp.float32),
        ],
    )(inputs, noise, wm_t, wv_t, bm, bv)
    return (v, o2)


# speedup vs baseline: 7.7921x; 1.0489x over previous
import math

import jax
import jax.numpy as jnp
from jax.experimental import pallas as pl

B = 16384
DIM = 128
BLOCK = 2048

_HALF_LOG_2PI = 0.5 * math.log(2.0 * math.pi)
_LOG2 = math.log(2.0)


def _fused_kernel(x_ref, nz_ref, wm_ref, wv_ref, bm_ref, bv_ref, v_ref, o2_ref):
    x = x_ref[...]
    nz = nz_ref[...]

    xb = x.astype(jnp.bfloat16)
    mean = jnp.dot(xb, wm_ref[...], preferred_element_type=jnp.float32) + bm_ref[...]
    log_var = jnp.dot(xb, wv_ref[...], preferred_element_type=jnp.float32) + bv_ref[...]

    std = jnp.exp(0.5 * log_var)
    u = nz * std + mean

    mx = jnp.max(x, axis=1, keepdims=True)
    iota = jax.lax.broadcasted_iota(jnp.int32, x.shape, 1).astype(jnp.float32)
    idx = jnp.min(jnp.where(x == mx, iota, float(DIM)), axis=1, keepdims=True)
    mask = iota == idx

    T = jnp.sum(jnp.where(mask, u, 0.0), axis=1, keepdims=True)

    d = T - u
    sp = jnp.maximum(d, 0.0) + jnp.log1p(jnp.exp(-jnp.abs(d)))

    v_ref[...] = jnp.where(mask, T, T - sp)

    log_det = jnp.sum(d - sp, axis=1, keepdims=True) + _LOG2
    log_pu = -0.5 * jnp.sum(nz * nz, axis=1, keepdims=True) - DIM * _HALF_LOG_2PI
    o2_ref[...] = log_det - log_pu


def kernel(inputs, W, b, noise):
    wm_t = W[:DIM].T.astype(jnp.bfloat16)
    wv_t = W[DIM:].T.astype(jnp.bfloat16)
    bm = b[:DIM].reshape(1, DIM)
    bv = b[DIM:].reshape(1, DIM)

    n = inputs.shape[0]
    grid = (n // BLOCK,)

    row_spec = pl.BlockSpec((BLOCK, DIM), lambda i: (i, 0))
    full_spec = pl.BlockSpec((DIM, DIM), lambda i: (0, 0))
    bias_spec = pl.BlockSpec((1, DIM), lambda i: (0, 0))

    v, o2 = pl.pallas_call(
        _fused_kernel,
        grid=grid,
        in_specs=[row_spec, row_spec, full_spec, full_spec, bias_spec, bias_spec],
        out_specs=[
            pl.BlockSpec((BLOCK, DIM), lambda i: (i, 0)),
            pl.BlockSpec((BLOCK, 1), lambda i: (i, 0)),
        ],
        out_shape=[
            jax.ShapeDtypeStruct((n, DIM), jnp.float32),
            jax.ShapeDtypeStruct((n, 1), jnp.float32),
        ],
    )(inputs, noise, wm_t, wv_t, bm, bv)
    return (v, o2)
